# baseline (speedup 1.0000x reference)
import jax
import jax.numpy as jnp
from jax.experimental import pallas as pl
from jax.experimental.pallas import tpu as pltpu

B, H, S, D = 8, 16, 4096, 128
Q = 32
P0 = 1024
E = P0 + Q


def _copy_body(kn, vn, kc, vc, ok, ov, sk1, sk2, sv1, sv2):
    ck = pltpu.make_async_copy(kc.at[:, :, pl.ds(0, P0), :],
                               ok.at[:, :, pl.ds(0, P0), :], sk1)
    cn = pltpu.make_async_copy(kn, ok.at[:, :, pl.ds(P0, Q), :], sk2)
    cv = pltpu.make_async_copy(vc.at[:, :, pl.ds(0, P0), :],
                               ov.at[:, :, pl.ds(0, P0), :], sv1)
    dn = pltpu.make_async_copy(vn, ov.at[:, :, pl.ds(P0, Q), :], sv2)
    ck.start()
    cn.start()
    cv.start()
    dn.start()
    ck.wait()
    cn.wait()
    cv.wait()
    dn.wait()


def kernel(k_new, v_new, k_cache, v_cache, start_pos):
    del start_pos
    out = pl.pallas_call(
        _copy_body,
        out_shape=[jax.ShapeDtypeStruct((B, H, E, D), jnp.float32)] * 2,
        in_specs=[pl.BlockSpec(memory_space=pl.ANY)] * 4,
        out_specs=[pl.BlockSpec(memory_space=pl.ANY)] * 2,
        scratch_shapes=[pltpu.SemaphoreType.DMA] * 4,
    )(k_new, v_new, k_cache, v_cache)
    return out[0], out[1]


# speedup vs baseline: 31.6041x; 31.6041x over previous
import jax
import jax.numpy as jnp
from jax.experimental import pallas as pl

B, H, S, D = 8, 16, 4096, 128
Q = 32
P0 = 1024
E = P0 + Q
BH = B * H


def _panel_body(kc, kn, vc, vn, ok, ov):
    ok[0, : P0, :] = kc[0]
    ok[0, P0:, :] = kn[0]
    ov[0, : P0, :] = vc[0]
    ov[0, P0:, :] = vn[0]


def kernel(k_new, v_new, k_cache, v_cache, start_pos):
    del start_pos
    kc = k_cache.reshape(BH, S, D)
    vc = v_cache.reshape(BH, S, D)
    kn = k_new.reshape(BH, Q, D)
    vn = v_new.reshape(BH, Q, D)
    ok, ov = pl.pallas_call(
        _panel_body,
        grid=(BH,),
        out_shape=[jax.ShapeDtypeStruct((BH, E, D), jnp.float32)] * 2,
        in_specs=[
            pl.BlockSpec((1, P0, D), lambda i: (i, 0, 0)),
            pl.BlockSpec((1, Q, D), lambda i: (i, 0, 0)),
            pl.BlockSpec((1, P0, D), lambda i: (i, 0, 0)),
            pl.BlockSpec((1, Q, D), lambda i: (i, 0, 0)),
        ],
        out_specs=[pl.BlockSpec((1, E, D), lambda i: (i, 0, 0))] * 2,
    )(kc, kn, vc, vn)
    return ok.reshape(B, H, E, D), ov.reshape(B, H, E, D)


# speedup vs baseline: 35.3202x; 1.1176x over previous
import functools

import jax
import jax.numpy as jnp
from jax import lax
from jax.experimental import pallas as pl
from jax.experimental.pallas import tpu as pltpu
from jax.experimental.pallas import tpu_sc as plsc

B, H, S, D = 8, 16, 4096, 128
Q = 32
P0 = 1024
E = P0 + Q
BH = B * H

NC, NS = 2, 16
NW = NC * NS
PPW = BH // NW
CHUNK = 256
NCHUNK = P0 // CHUNK

_mesh = plsc.VectorSubcoreMesh(
    core_axis_name="c", subcore_axis_name="s", num_cores=NC, num_subcores=NS)


def _sc_body(kc, kn, vc, vn, ok, ov, buf0, buf1, l0, l1, s0, s1):
    c = lax.axis_index("c")
    s = lax.axis_index("s")
    base = (s * NC + c) * PPW

    bufs = (buf0, buf1)
    lsems = (l0, l1)
    ssems = (s0, s1)

    jobs = []
    for p_local in range(PPW):
        for which in range(2):
            for ci in range(NCHUNK):
                jobs.append((p_local, which, ci))
            jobs.append((p_local, which, None))

    tensors = ((kc, kn, ok), (vc, vn, ov))

    def mk(g):
        p_local, which, ci = jobs[g]
        tin, tnew, tout = tensors[which]
        p = base + p_local
        b = g % 2
        if ci is None:
            src = tnew.at[p]
            dst = tout.at[p, pl.ds(P0, Q), :]
            rows = Q
        else:
            src = tin.at[p, pl.ds(ci * CHUNK, CHUNK), :]
            dst = tout.at[p, pl.ds(ci * CHUNK, CHUNK), :]
            rows = CHUNK
        ld = pltpu.make_async_copy(src, bufs[b].at[pl.ds(0, rows)], lsems[b])
        st = pltpu.make_async_copy(bufs[b].at[pl.ds(0, rows)], dst, ssems[b])
        return ld, st

    n = len(jobs)
    prev_store = [None, None]
    pending = None
    for g in range(n):
        b = g % 2
        ld, st = mk(g)
        if prev_store[b] is not None:
            prev_store[b].wait()
        ld.start()
        if pending is not None:
            pld, pst = pending
            pld.wait()
            pst.start()
            prev_store[(g - 1) % 2] = pst
        pending = (ld, st)
    pld, pst = pending
    pld.wait()
    pst.start()
    prev_store[(n - 1) % 2] = pst
    for d in prev_store:
        if d is not None:
            d.wait()


@functools.partial(
    pl.kernel,
    out_type=[jax.ShapeDtypeStruct((BH, E, D), jnp.float32)] * 2,
    mesh=_mesh,
    scratch_types=[
        pltpu.VMEM((CHUNK, D), jnp.float32),
        pltpu.VMEM((CHUNK, D), jnp.float32),
        pltpu.SemaphoreType.DMA,
        pltpu.SemaphoreType.DMA,
        pltpu.SemaphoreType.DMA,
        pltpu.SemaphoreType.DMA,
    ],
)
def _sc_copy(kc, kn, vc, vn, ok, ov, buf0, buf1, l0, l1, s0, s1):
    _sc_body(kc, kn, vc, vn, ok, ov, buf0, buf1, l0, l1, s0, s1)


def kernel(k_new, v_new, k_cache, v_cache, start_pos):
    del start_pos
    kc = k_cache.reshape(BH, S, D)
    vc = v_cache.reshape(BH, S, D)
    kn = k_new.reshape(BH, Q, D)
    vn = v_new.reshape(BH, Q, D)
    ok, ov = _sc_copy(kc, kn, vc, vn)
    return ok.reshape(B, H, E, D), ov.reshape(B, H, E, D)
